# baseline (speedup 1.0000x reference)
import functools

import jax
import jax.numpy as jnp
from jax import lax
from jax.experimental import pallas as pl
from jax.experimental.pallas import tpu as pltpu
from jax.experimental.pallas import tpu_sc as plsc

_TC_R = 256


def _sc_gather(table, index):
    N, C = table.shape
    B = index.shape[0]
    info = plsc.get_sparse_core_info()
    nw = info.num_cores * info.num_subcores
    b_per_w = B // nw
    mesh = plsc.VectorSubcoreMesh(core_axis_name="c", subcore_axis_name="s")

    @functools.partial(
        pl.kernel,
        mesh=mesh,
        out_type=jax.ShapeDtypeStruct((B, C), jnp.float32),
        scratch_types=[
            pltpu.VMEM((b_per_w,), jnp.int32),
            pltpu.VMEM((b_per_w, C), jnp.float32),
            pltpu.SemaphoreType.DMA,
        ],
        compiler_params=pltpu.CompilerParams(use_tc_tiling_on_sc=False),
    )
    def gather(table_hbm, idx_hbm, out_hbm, idx_v, rows_v, sem):
        cid = lax.axis_index("c")
        sid = lax.axis_index("s")
        wid = sid * info.num_cores + cid
        base = wid * b_per_w
        pltpu.sync_copy(idx_hbm.at[pl.ds(base, b_per_w)], idx_v)
        pltpu.async_copy(table_hbm.at[idx_v], rows_v, sem).wait()
        pltpu.sync_copy(rows_v, out_hbm.at[pl.ds(base, b_per_w)])

    return gather(table, index)


def _tc_body(x_ref, g_ref, loss_ref, acc_ref):
    i = pl.program_id(0)
    nsteps = pl.num_programs(0)

    x = x_ref[...]
    g = g_ref[...]
    m = jnp.max(x, axis=1, keepdims=True)
    lse = m + jnp.log(jnp.sum(jnp.exp(x - m), axis=1, keepdims=True))
    s1 = jnp.sum(g, axis=1, keepdims=True)
    d = jnp.sum(x * g, axis=1, keepdims=True)
    loss = lse * s1 - d
    loss_ref[...] = loss

    @pl.when(i == 0)
    def _():
        acc_ref[...] = jnp.zeros_like(acc_ref)

    total = acc_ref[...] + jnp.sum(loss).reshape(1, 1)
    acc_ref[...] = total

    @pl.when(i == nsteps - 1)
    def _():
        acc_ref[...] = total / (nsteps * _TC_R)


def kernel(outputs, index, confidence):
    B, C = outputs.shape
    G = B // _TC_R
    gathered = _sc_gather(confidence, index)
    loss2, acc = pl.pallas_call(
        _tc_body,
        grid=(G,),
        in_specs=[
            pl.BlockSpec((_TC_R, C), lambda i: (i, 0)),
            pl.BlockSpec((_TC_R, C), lambda i: (i, 0)),
        ],
        out_specs=[
            pl.BlockSpec((_TC_R, 1), lambda i: (i, 0)),
            pl.BlockSpec((1, 1), lambda i: (0, 0)),
        ],
        out_shape=[
            jax.ShapeDtypeStruct((B, 1), jnp.float32),
            jax.ShapeDtypeStruct((1, 1), jnp.float32),
        ],
    )(outputs, gathered)
    return (acc[0, 0], loss2.reshape(B))


# speedup vs baseline: 4.8535x; 4.8535x over previous
import jax
import jax.numpy as jnp
from jax.experimental import pallas as pl
from jax.experimental.pallas import tpu as pltpu

_R = 256


def _issue_block(idx_ref, conf_hbm, buf, sem, step):
    base = step * _R

    def issue_one(k, carry):
        row = idx_ref[base + k]
        pltpu.make_async_copy(
            conf_hbm.at[pl.ds(row, 1), :],
            buf.at[pl.ds(k, 1), :],
            sem,
        ).start()
        return carry

    jax.lax.fori_loop(0, _R, issue_one, 0, unroll=8)


def _wait_block(conf_hbm, buf, sem):
    def wait_one(k, carry):
        pltpu.make_async_copy(
            conf_hbm.at[pl.ds(0, 1), :],
            buf.at[pl.ds(0, 1), :],
            sem,
        ).wait()
        return carry

    jax.lax.fori_loop(0, _R, wait_one, 0, unroll=8)


def _body(idx_ref, x_ref, conf_hbm, loss_ref, acc_ref, buf, sem):
    i = pl.program_id(0)
    nsteps = pl.num_programs(0)
    par = jax.lax.rem(i, 2)
    nxt = jax.lax.rem(i + 1, 2)

    @pl.when(i == 0)
    def _():
        _issue_block(idx_ref, conf_hbm, buf.at[0], sem.at[0], 0)

    @pl.when(i + 1 < nsteps)
    def _():
        _issue_block(idx_ref, conf_hbm, buf.at[nxt], sem.at[nxt], i + 1)

    _wait_block(conf_hbm, buf.at[par], sem.at[par])

    x = x_ref[...]
    g = buf[par]
    m = jnp.max(x, axis=1, keepdims=True)
    lse = m + jnp.log(jnp.sum(jnp.exp(x - m), axis=1, keepdims=True))
    s1 = jnp.sum(g, axis=1, keepdims=True)
    d = jnp.sum(x * g, axis=1, keepdims=True)
    loss = lse * s1 - d
    loss_ref[...] = loss

    @pl.when(i == 0)
    def _():
        acc_ref[...] = jnp.zeros_like(acc_ref)

    total = acc_ref[...] + jnp.sum(loss).reshape(1, 1)
    acc_ref[...] = total

    @pl.when(i == nsteps - 1)
    def _():
        acc_ref[...] = total / (nsteps * _R)


def kernel(outputs, index, confidence):
    B, C = outputs.shape
    G = B // _R
    grid_spec = pltpu.PrefetchScalarGridSpec(
        num_scalar_prefetch=1,
        grid=(G,),
        in_specs=[
            pl.BlockSpec((_R, C), lambda i, idx: (i, 0)),
            pl.BlockSpec(memory_space=pl.ANY),
        ],
        out_specs=[
            pl.BlockSpec((_R, 1), lambda i, idx: (i, 0)),
            pl.BlockSpec((1, 1), lambda i, idx: (0, 0)),
        ],
        scratch_shapes=[
            pltpu.VMEM((2, _R, C), jnp.float32),
            pltpu.SemaphoreType.DMA((2,)),
        ],
    )
    loss2, acc = pl.pallas_call(
        _body,
        grid_spec=grid_spec,
        out_shape=[
            jax.ShapeDtypeStruct((B, 1), jnp.float32),
            jax.ShapeDtypeStruct((1, 1), jnp.float32),
        ],
    )(index, outputs, confidence)
    return (acc[0, 0], loss2.reshape(B))
